# Initial kernel scaffold; baseline (speedup 1.0000x reference)
import jax
import jax.numpy as jnp
from jax.experimental import pallas as pl


def kernel(x, edge_index, W1, b1, W2, b2):
    raise NotImplementedError("write your pallas kernel here")


# speedup vs baseline: 7.8812x; 7.8812x over previous
import functools

import jax
import jax.numpy as jnp
from jax import lax
from jax.experimental import pallas as pl
from jax.experimental.pallas import tpu as pltpu
from jax.experimental.pallas import tpu_sc as plsc

NC = 2
NS = 16
LANES = 16
F32 = jnp.float32


def _chunk_size(per_tile):
    for c in range(128, 0, -8):
        if per_tile % c == 0:
            return c
    raise ValueError(per_tile)


def _half_rows(N):
    return -(-N // (2 * NS * 400)) * NS * 400


def _zero_shared(acc_sh, zbuf, sid, n_rows_tile):
    zr = zbuf.shape[0]
    zeros16 = jnp.zeros((LANES,), F32)

    def zb(i, _):
        zbuf[i, :] = zeros16
        return _

    lax.fori_loop(0, zr, zb, None)
    r0 = sid * n_rows_tile

    def cp(i, _):
        pltpu.sync_copy(zbuf, acc_sh.at[pl.ds(r0 + i * zr, zr)])
        return _

    lax.fori_loop(0, n_rows_tile // zr, cp, None)


def _redirect(sidx_v, redir_v, base_node, half, chunk):
    for k in range(chunk // LANES):
        v = sidx_v[pl.ds(k * LANES, LANES)]
        local = v - base_node
        owned = (local >= 0) & (local < half)
        redir_v[pl.ds(k * LANES, LANES)] = jnp.where(owned, local,
                                                     jnp.int32(half))


def _make_edge_kernel(E, N, gather):
    ept = E // NS
    chunk = _chunk_size(ept)
    half = _half_rows(N)
    n_rows_tile = half // NS
    zr = 400
    mesh = plsc.VectorSubcoreMesh(core_axis_name="c", subcore_axis_name="s")

    scratch = [
        pltpu.VMEM((chunk,), jnp.int32),
        pltpu.VMEM((chunk,), jnp.int32),
        pltpu.VMEM((chunk, LANES), F32),
        pltpu.VMEM((zr, LANES), F32),
        pltpu.VMEM_SHARED((half + 8, LANES), F32),
    ]
    if gather:
        scratch.insert(0, pltpu.VMEM((chunk,), jnp.int32))
        scratch.append(pltpu.SemaphoreType.DMA)

    @functools.partial(
        pl.kernel,
        out_type=jax.ShapeDtypeStruct((NC * half, LANES), F32),
        mesh=mesh,
        scratch_types=scratch,
        compiler_params=pltpu.CompilerParams(use_tc_tiling_on_sc=False),
    )
    def edge_kernel(*refs):
        if gather:
            (gidx_hbm, sidx_hbm, table_hbm, out_hbm,
             gidx_v, sidx_v, redir_v, rows_v, zbuf, acc_sh, sem) = refs
        else:
            (sidx_hbm, out_hbm,
             sidx_v, redir_v, rows_v, zbuf, acc_sh) = refs
        cid = lax.axis_index("c")
        sid = lax.axis_index("s")
        base_node = cid * half
        _zero_shared(acc_sh, zbuf, sid, n_rows_tile)

        if not gather:
            ones16 = jnp.ones((LANES,), F32)

            def fill(i, _):
                rows_v[i, :] = ones16
                return _

            lax.fori_loop(0, chunk, fill, None)
        plsc.subcore_barrier()

        e0 = sid * ept

        def step(j, _):
            base = e0 + j * chunk
            pltpu.sync_copy(sidx_hbm.at[pl.ds(base, chunk)], sidx_v)
            _redirect(sidx_v, redir_v, base_node, half, chunk)
            if gather:
                pltpu.sync_copy(gidx_hbm.at[pl.ds(base, chunk)], gidx_v)
                pltpu.async_copy(table_hbm.at[gidx_v], rows_v, sem).wait()
            pltpu.sync_copy(rows_v, acc_sh.at[redir_v], add=True)
            return _

        lax.fori_loop(0, ept // chunk, step, None)
        plsc.subcore_barrier()
        r0 = sid * n_rows_tile
        pltpu.sync_copy(acc_sh.at[pl.ds(r0, n_rows_tile)],
                        out_hbm.at[pl.ds(base_node + r0, n_rows_tile)])

    return edge_kernel


def _tc_mm(x, W1, deg16, blk=1000):
    N, F = x.shape
    H = W1.shape[1]

    def body(x_ref, w_ref, deg_ref, g_ref, dinv_ref):
        h = jnp.dot(x_ref[...], w_ref[...], preferred_element_type=F32)
        dinv = 1.0 / jnp.sqrt(deg_ref[...] + 1.0)
        g_ref[...] = h * dinv
        dinv_ref[...] = dinv

    return pl.pallas_call(
        body,
        grid=(N // blk,),
        in_specs=[
            pl.BlockSpec((blk, F), lambda i: (i, 0)),
            pl.BlockSpec((F, H), lambda i: (0, 0)),
            pl.BlockSpec((blk, LANES), lambda i: (i, 0)),
        ],
        out_specs=[
            pl.BlockSpec((blk, H), lambda i: (i, 0)),
            pl.BlockSpec((blk, LANES), lambda i: (i, 0)),
        ],
        out_shape=[
            jax.ShapeDtypeStruct((N, H), F32),
            jax.ShapeDtypeStruct((N, LANES), F32),
        ],
    )(x, W1, deg16)


def _tc_final(agg, csum, g, dinv16, b1, W2, b2, blk=1000):
    N = g.shape[0]
    H = g.shape[1]
    EMB = W2.shape[1]

    def body(agg_ref, csum_ref, g_ref, dinv_ref, b1_ref, w2_ref, b2_ref,
             out_ref, acc_ref):
        i = pl.program_id(0)

        @pl.when(i == 0)
        def _():
            acc_ref[...] = jnp.zeros_like(acc_ref)

        dinv = dinv_ref[...]
        r = jnp.maximum(
            dinv * (agg_ref[...] + g_ref[...]) + b1_ref[...], 0.0)
        c16 = dinv * (csum_ref[...] + dinv)
        acc_ref[...] += jnp.sum(c16 * r, axis=0, keepdims=True)

        @pl.when(i == pl.num_programs(0) - 1)
        def _():
            acc = acc_ref[...]
            out_ref[...] = (jnp.dot(acc, w2_ref[...],
                                    preferred_element_type=F32) / N
                            + b2_ref[...])

    return pl.pallas_call(
        body,
        grid=(N // blk,),
        in_specs=[
            pl.BlockSpec((blk, H), lambda i: (i, 0)),
            pl.BlockSpec((blk, LANES), lambda i: (i, 0)),
            pl.BlockSpec((blk, H), lambda i: (i, 0)),
            pl.BlockSpec((blk, LANES), lambda i: (i, 0)),
            pl.BlockSpec((1, H), lambda i: (0, 0)),
            pl.BlockSpec((H, EMB), lambda i: (0, 0)),
            pl.BlockSpec((1, EMB), lambda i: (0, 0)),
        ],
        out_specs=pl.BlockSpec((1, EMB), lambda i: (0, 0)),
        out_shape=jax.ShapeDtypeStruct((1, EMB), F32),
        scratch_shapes=[pltpu.VMEM((1, LANES), F32)],
    )(agg, csum, g, dinv16, b1, W2, b2)


def kernel(x, edge_index, W1, b1, W2, b2):
    N = x.shape[0]
    E = edge_index.shape[1]
    src = edge_index[0]
    dst = edge_index[1]

    deg16 = _make_edge_kernel(E, N, gather=False)(dst)
    g, dinv16 = _tc_mm(x, W1, deg16)
    edge_k = _make_edge_kernel(E, N, gather=True)
    agg = edge_k(src, dst, g)
    csum = edge_k(dst, src, dinv16)
    return _tc_final(agg, csum, g, dinv16,
                     b1.reshape(1, -1), W2, b2.reshape(1, -1))


# speedup vs baseline: 13.5444x; 1.7186x over previous
import functools

import jax
import jax.numpy as jnp
from jax import lax
from jax.experimental import pallas as pl
from jax.experimental.pallas import tpu as pltpu
from jax.experimental.pallas import tpu_sc as plsc

NC = 2
NS = 16
LANES = 16
F32 = jnp.float32
NBUF = 4


def _chunk_size(per_tile):
    for c in range(128, 0, -8):
        if per_tile % c == 0:
            return c
    raise ValueError(per_tile)


def _half_rows(N):
    return -(-N // (2 * NS * 400)) * NS * 400


def _zero_shared(acc_sh, zbuf, sid, n_rows_tile):
    zr = zbuf.shape[0]
    zeros16 = jnp.zeros((LANES,), F32)

    def zb(i, _):
        zbuf[i, :] = zeros16
        return _

    lax.fori_loop(0, zr, zb, None)
    r0 = sid * n_rows_tile

    def cp(i, _):
        pltpu.sync_copy(zbuf, acc_sh.at[pl.ds(r0 + i * zr, zr)])
        return _

    lax.fori_loop(0, n_rows_tile // zr, cp, None)


def _make_edge_kernel(E, N, gather, gidx_row=0):
    sidx_row = 1 - gidx_row if gather else 1
    ept = E // NS
    chunk = _chunk_size(ept)
    niter = ept // chunk
    half = _half_rows(N)
    n_rows_tile = half // NS
    zr = 400
    mesh = plsc.VectorSubcoreMesh(core_axis_name="c", subcore_axis_name="s")

    scratch = [
        pltpu.VMEM((NBUF, 2, chunk), jnp.int32),
        pltpu.VMEM((NBUF, chunk), jnp.int32),
        pltpu.VMEM((NBUF, chunk, LANES), F32) if gather
        else pltpu.VMEM((chunk, LANES), F32),
        pltpu.VMEM((zr, LANES), F32),
        pltpu.VMEM_SHARED((half + 8, LANES), F32),
        pltpu.SemaphoreType.DMA,
        pltpu.SemaphoreType.DMA,
    ]
    if gather:
        scratch.append(pltpu.SemaphoreType.DMA)

    @functools.partial(
        pl.kernel,
        out_type=jax.ShapeDtypeStruct((NC * half, LANES), F32),
        mesh=mesh,
        scratch_types=scratch,
        compiler_params=pltpu.CompilerParams(use_tc_tiling_on_sc=False),
    )
    def edge_kernel(*refs):
        if gather:
            (ei_hbm, table_hbm, out_hbm,
             idxb, redir, rows, zbuf, acc_sh, semI, semS, semG) = refs
        else:
            (ei_hbm, out_hbm,
             idxb, redir, rows, zbuf, acc_sh, semI, semS) = refs
        cid = lax.axis_index("c")
        sid = lax.axis_index("s")
        base_node = cid * half
        dummy = jnp.int32(half)

        e0 = sid * ept

        def fire_idx(j, b):
            pltpu.async_copy(
                ei_hbm.at[:, pl.ds(e0 + j * chunk, chunk)], idxb.at[b], semI)

        def wait_idx(b):
            pltpu.make_async_copy(
                ei_hbm.at[:, pl.ds(0, chunk)], idxb.at[b], semI).wait()

        def redirect(b):
            for k in range(chunk // LANES):
                v = idxb[b, sidx_row, pl.ds(k * LANES, LANES)]
                local = v - base_node
                owned = (local >= 0) & (local < half)
                redir[b, pl.ds(k * LANES, LANES)] = jnp.where(
                    owned, local, dummy)

        def fire_gather(b):
            pltpu.async_copy(table_hbm.at[idxb.at[b, gidx_row]],
                             rows.at[b], semG)

        def wait_gather(b):
            pltpu.make_async_copy(table_hbm.at[pl.ds(0, chunk)],
                                  rows.at[b], semG).wait()

        def fire_scatter(b):
            src = rows.at[b] if gather else rows
            pltpu.async_copy(src, acc_sh.at[redir.at[b]], semS, add=True)

        def wait_scatter(b):
            src = rows.at[b] if gather else rows
            pltpu.make_async_copy(table_hbm.at[pl.ds(0, chunk)] if gather
                                  else out_hbm.at[pl.ds(0, chunk)],
                                  src, semS).wait()

        def step(j, b, first4, warm, alive, tail):
            if alive:
                wait_idx(b)
            if not first4:
                wait_scatter(b)
            if alive:
                redirect(b)
                if gather:
                    fire_gather(b)
            if warm:
                b2 = (b - 2) % NBUF
                if gather:
                    wait_gather(b2)
                fire_scatter(b2)
            if tail:
                fire_idx(j + 2, (b + 2) % NBUF)

        fire_idx(0, 0)
        fire_idx(1, 1)
        if not gather:
            ones16 = jnp.ones((LANES,), F32)

            def fill(i, _):
                rows[i, :] = ones16
                return _

            lax.fori_loop(0, chunk, fill, None)
        _zero_shared(acc_sh, zbuf, sid, n_rows_tile)
        plsc.subcore_barrier()

        for j in range(4):
            step(j, j, True, j >= 2, True, True)

        G = (niter - 2 - 4) // 4

        def body(g, _):
            j0 = 4 * (g + 1)
            for b in range(4):
                step(j0 + b, b, False, True, True, True)
            return _

        lax.fori_loop(0, G, body, None)

        for j in range(4 + 4 * G, niter + NBUF):
            step(j, j % NBUF, False, 2 <= j and j - 2 < niter,
                 j < niter, j + 2 < niter)

        plsc.subcore_barrier()
        r0 = sid * n_rows_tile
        pltpu.sync_copy(acc_sh.at[pl.ds(r0, n_rows_tile)],
                        out_hbm.at[pl.ds(base_node + r0, n_rows_tile)])

    return edge_kernel


def _tc_mm(x, W1, deg16, blk=1000):
    N, F = x.shape
    H = W1.shape[1]

    def body(x_ref, w_ref, deg_ref, g_ref, dinv_ref):
        h = jnp.dot(x_ref[...], w_ref[...], preferred_element_type=F32)
        dinv = 1.0 / jnp.sqrt(deg_ref[...] + 1.0)
        g_ref[...] = h * dinv
        dinv_ref[...] = dinv

    return pl.pallas_call(
        body,
        grid=(N // blk,),
        in_specs=[
            pl.BlockSpec((blk, F), lambda i: (i, 0)),
            pl.BlockSpec((F, H), lambda i: (0, 0)),
            pl.BlockSpec((blk, LANES), lambda i: (i, 0)),
        ],
        out_specs=[
            pl.BlockSpec((blk, H), lambda i: (i, 0)),
            pl.BlockSpec((blk, LANES), lambda i: (i, 0)),
        ],
        out_shape=[
            jax.ShapeDtypeStruct((N, H), F32),
            jax.ShapeDtypeStruct((N, LANES), F32),
        ],
    )(x, W1, deg16)


def _tc_final(agg, csum, g, dinv16, b1, W2, b2, blk=1000):
    N = g.shape[0]
    H = g.shape[1]
    EMB = W2.shape[1]

    def body(agg_ref, csum_ref, g_ref, dinv_ref, b1_ref, w2_ref, b2_ref,
             out_ref, acc_ref):
        i = pl.program_id(0)

        @pl.when(i == 0)
        def _():
            acc_ref[...] = jnp.zeros_like(acc_ref)

        dinv = dinv_ref[...]
        r = jnp.maximum(
            dinv * (agg_ref[...] + g_ref[...]) + b1_ref[...], 0.0)
        c16 = dinv * (csum_ref[...] + dinv)
        acc_ref[...] += jnp.sum(c16 * r, axis=0, keepdims=True)

        @pl.when(i == pl.num_programs(0) - 1)
        def _():
            acc = acc_ref[...]
            out_ref[...] = (jnp.dot(acc, w2_ref[...],
                                    preferred_element_type=F32) / N
                            + b2_ref[...])

    return pl.pallas_call(
        body,
        grid=(N // blk,),
        in_specs=[
            pl.BlockSpec((blk, H), lambda i: (i, 0)),
            pl.BlockSpec((blk, LANES), lambda i: (i, 0)),
            pl.BlockSpec((blk, H), lambda i: (i, 0)),
            pl.BlockSpec((blk, LANES), lambda i: (i, 0)),
            pl.BlockSpec((1, H), lambda i: (0, 0)),
            pl.BlockSpec((H, EMB), lambda i: (0, 0)),
            pl.BlockSpec((1, EMB), lambda i: (0, 0)),
        ],
        out_specs=pl.BlockSpec((1, EMB), lambda i: (0, 0)),
        out_shape=jax.ShapeDtypeStruct((1, EMB), F32),
        scratch_shapes=[pltpu.VMEM((1, LANES), F32)],
    )(agg, csum, g, dinv16, b1, W2, b2)


def kernel(x, edge_index, W1, b1, W2, b2):
    N = x.shape[0]
    E = edge_index.shape[1]

    deg16 = _make_edge_kernel(E, N, gather=False)(edge_index)
    g, dinv16 = _tc_mm(x, W1, deg16)
    agg = _make_edge_kernel(E, N, gather=True, gidx_row=0)(edge_index, g)
    csum = _make_edge_kernel(E, N, gather=True, gidx_row=1)(edge_index,
                                                            dinv16)
    return _tc_final(agg, csum, g, dinv16,
                     b1.reshape(1, -1), W2, b2.reshape(1, -1))


# speedup vs baseline: 24.3528x; 1.7980x over previous
import functools

import jax
import jax.numpy as jnp
from jax import lax
from jax.experimental import pallas as pl
from jax.experimental.pallas import tpu as pltpu
from jax.experimental.pallas import tpu_sc as plsc

NC = 2
NS = 16
LANES = 16
F32 = jnp.float32
NBUF = 4


def _chunk_size(per_tile):
    for c in range(128, 0, -8):
        if per_tile % c == 0:
            return c
    raise ValueError(per_tile)


def _half_rows(N):
    return -(-N // (2 * NS * 400)) * NS * 400


def _zero_shared(acc_sh, zbuf, sid, n_rows_tile):
    zr = zbuf.shape[0]
    zeros16 = jnp.zeros((LANES,), F32)

    def zb(i, _):
        zbuf[i, :] = zeros16
        return _

    lax.fori_loop(0, zr, zb, None)
    r0 = sid * n_rows_tile

    def cp(i, _):
        pltpu.sync_copy(zbuf, acc_sh.at[pl.ds(r0 + i * zr, zr)])
        return _

    lax.fori_loop(0, n_rows_tile // zr, cp, None)


def _make_edge_kernel(E, N, gather, gidx_row=0):
    sidx_row = 1 - gidx_row if gather else 1
    ept = E // NS
    chunk = _chunk_size(ept)
    niter = ept // chunk
    half = _half_rows(N)
    n_rows_tile = half // NS
    zr = 400
    mesh = plsc.VectorSubcoreMesh(core_axis_name="c", subcore_axis_name="s")

    scratch = [
        pltpu.VMEM((NBUF, 2, chunk), jnp.int32),
        pltpu.VMEM((NBUF, chunk), jnp.int32),
        pltpu.VMEM((NBUF, chunk, LANES), F32) if gather
        else pltpu.VMEM((chunk, LANES), F32),
        pltpu.VMEM((zr, LANES), F32),
        pltpu.VMEM_SHARED((half + 8, LANES), F32),
        pltpu.SemaphoreType.DMA,
        pltpu.SemaphoreType.DMA,
    ]
    if gather:
        scratch.append(pltpu.SemaphoreType.DMA)

    @functools.partial(
        pl.kernel,
        out_type=jax.ShapeDtypeStruct((NC * half, LANES), F32),
        mesh=mesh,
        scratch_types=scratch,
        compiler_params=pltpu.CompilerParams(use_tc_tiling_on_sc=False),
    )
    def edge_kernel(*refs):
        if gather:
            (ei_hbm, table_hbm, out_hbm,
             idxb, redir, rows, zbuf, acc_sh, semI, semS, semG) = refs
        else:
            (ei_hbm, out_hbm,
             idxb, redir, rows, zbuf, acc_sh, semI, semS) = refs
        cid = lax.axis_index("c")
        sid = lax.axis_index("s")
        base_node = cid * half
        dummy = jnp.int32(half) + (lax.iota(jnp.int32, LANES) & 7)

        e0 = sid * ept

        def fire_idx(j, b):
            pltpu.async_copy(
                ei_hbm.at[:, pl.ds(e0 + j * chunk, chunk)], idxb.at[b], semI)

        def wait_idx(b):
            pltpu.make_async_copy(
                ei_hbm.at[:, pl.ds(0, chunk)], idxb.at[b], semI).wait()

        def redirect(b):
            for k in range(chunk // LANES):
                v = idxb[b, sidx_row, pl.ds(k * LANES, LANES)]
                local = v - base_node
                owned = (local >= 0) & (local < half)
                redir[b, pl.ds(k * LANES, LANES)] = jnp.where(
                    owned, local, dummy)

        def fire_gather(b):
            pltpu.async_copy(table_hbm.at[idxb.at[b, gidx_row]],
                             rows.at[b], semG)

        def wait_gather(b):
            pltpu.make_async_copy(table_hbm.at[pl.ds(0, chunk)],
                                  rows.at[b], semG).wait()

        def fire_scatter(b):
            src = rows.at[b] if gather else rows
            pltpu.async_copy(src, acc_sh.at[redir.at[b]], semS, add=True)

        def wait_scatter(b):
            src = rows.at[b] if gather else rows
            pltpu.make_async_copy(table_hbm.at[pl.ds(0, chunk)] if gather
                                  else out_hbm.at[pl.ds(0, chunk)],
                                  src, semS).wait()

        def step(j, b, first4, warm, alive, tail):
            if alive:
                wait_idx(b)
            if not first4:
                wait_scatter(b)
            if alive:
                redirect(b)
                if gather:
                    fire_gather(b)
            if warm:
                b2 = (b - 2) % NBUF
                if gather:
                    wait_gather(b2)
                fire_scatter(b2)
            if tail:
                fire_idx(j + 2, (b + 2) % NBUF)

        fire_idx(0, 0)
        fire_idx(1, 1)
        if not gather:
            ones16 = jnp.ones((LANES,), F32)

            def fill(i, _):
                rows[i, :] = ones16
                return _

            lax.fori_loop(0, chunk, fill, None)
        _zero_shared(acc_sh, zbuf, sid, n_rows_tile)
        plsc.subcore_barrier()

        for j in range(4):
            step(j, j, True, j >= 2, True, True)

        G = (niter - 2 - 4) // 4

        def body(g, _):
            j0 = 4 * (g + 1)
            for b in range(4):
                step(j0 + b, b, False, True, True, True)
            return _

        lax.fori_loop(0, G, body, None)

        for j in range(4 + 4 * G, niter + NBUF):
            step(j, j % NBUF, False, 2 <= j and j - 2 < niter,
                 j < niter, j + 2 < niter)

        plsc.subcore_barrier()
        r0 = sid * n_rows_tile
        pltpu.sync_copy(acc_sh.at[pl.ds(r0, n_rows_tile)],
                        out_hbm.at[pl.ds(base_node + r0, n_rows_tile)])

    return edge_kernel


def _tc_matmul(x, W1, blk=1000):
    N, F = x.shape
    H = W1.shape[1]

    def body(x_ref, w_ref, h_ref):
        h_ref[...] = jnp.dot(x_ref[...], w_ref[...],
                             preferred_element_type=F32)

    return pl.pallas_call(
        body,
        grid=(N // blk,),
        in_specs=[
            pl.BlockSpec((blk, F), lambda i: (i, 0)),
            pl.BlockSpec((F, H), lambda i: (0, 0)),
        ],
        out_specs=pl.BlockSpec((blk, H), lambda i: (i, 0)),
        out_shape=jax.ShapeDtypeStruct((N, H), F32),
    )(x, W1)


def _tc_scale(h, deg16, blk=1000):
    N, H = h.shape

    def body(h_ref, deg_ref, g_ref, dinv_ref):
        dinv = 1.0 / jnp.sqrt(deg_ref[...] + 1.0)
        g_ref[...] = h_ref[...] * dinv
        dinv_ref[...] = dinv

    return pl.pallas_call(
        body,
        grid=(N // blk,),
        in_specs=[
            pl.BlockSpec((blk, H), lambda i: (i, 0)),
            pl.BlockSpec((blk, LANES), lambda i: (i, 0)),
        ],
        out_specs=[
            pl.BlockSpec((blk, H), lambda i: (i, 0)),
            pl.BlockSpec((blk, LANES), lambda i: (i, 0)),
        ],
        out_shape=[
            jax.ShapeDtypeStruct((N, H), F32),
            jax.ShapeDtypeStruct((N, LANES), F32),
        ],
    )(h, deg16)


def _tc_final(agg, csum, g, dinv16, b1, W2, b2, blk=1000):
    N = g.shape[0]
    H = g.shape[1]
    EMB = W2.shape[1]

    def body(agg_ref, csum_ref, g_ref, dinv_ref, b1_ref, w2_ref, b2_ref,
             out_ref, acc_ref):
        i = pl.program_id(0)

        @pl.when(i == 0)
        def _():
            acc_ref[...] = jnp.zeros_like(acc_ref)

        dinv = dinv_ref[...]
        r = jnp.maximum(
            dinv * (agg_ref[...] + g_ref[...]) + b1_ref[...], 0.0)
        c16 = dinv * (csum_ref[...] + dinv)
        acc_ref[...] += jnp.sum(c16 * r, axis=0, keepdims=True)

        @pl.when(i == pl.num_programs(0) - 1)
        def _():
            acc = acc_ref[...]
            out_ref[...] = (jnp.dot(acc, w2_ref[...],
                                    preferred_element_type=F32) / N
                            + b2_ref[...])

    return pl.pallas_call(
        body,
        grid=(N // blk,),
        in_specs=[
            pl.BlockSpec((blk, H), lambda i: (i, 0)),
            pl.BlockSpec((blk, LANES), lambda i: (i, 0)),
            pl.BlockSpec((blk, H), lambda i: (i, 0)),
            pl.BlockSpec((blk, LANES), lambda i: (i, 0)),
            pl.BlockSpec((1, H), lambda i: (0, 0)),
            pl.BlockSpec((H, EMB), lambda i: (0, 0)),
            pl.BlockSpec((1, EMB), lambda i: (0, 0)),
        ],
        out_specs=pl.BlockSpec((1, EMB), lambda i: (0, 0)),
        out_shape=jax.ShapeDtypeStruct((1, EMB), F32),
        scratch_shapes=[pltpu.VMEM((1, LANES), F32)],
    )(agg, csum, g, dinv16, b1, W2, b2)


def kernel(x, edge_index, W1, b1, W2, b2):
    N = x.shape[0]
    E = edge_index.shape[1]

    h = _tc_matmul(x, W1)
    deg16 = _make_edge_kernel(E, N, gather=False)(edge_index)
    g, dinv16 = _tc_scale(h, deg16)
    agg = _make_edge_kernel(E, N, gather=True, gidx_row=0)(edge_index, g)
    csum = _make_edge_kernel(E, N, gather=True, gidx_row=1)(edge_index,
                                                            dinv16)
    return _tc_final(agg, csum, g, dinv16,
                     b1.reshape(1, -1), W2, b2.reshape(1, -1))


# speedup vs baseline: 27.2533x; 1.1191x over previous
import functools

import jax
import jax.numpy as jnp
from jax import lax
from jax.experimental import pallas as pl
from jax.experimental.pallas import tpu as pltpu
from jax.experimental.pallas import tpu_sc as plsc

NC = 2
NS = 16
LANES = 16
F32 = jnp.float32
NBUF = 4


def _chunk_size(per_tile):
    for c in range(128, 0, -8):
        if per_tile % c == 0:
            return c
    raise ValueError(per_tile)


def _half_rows(N):
    return -(-N // (2 * NS * 400)) * NS * 400


def _zero_shared(acc_sh, zbuf, sid, n_rows_tile):
    zr = zbuf.shape[0]
    zeros16 = jnp.zeros((LANES,), F32)

    def zb(i, _):
        zbuf[i, :] = zeros16
        return _

    lax.fori_loop(0, zr, zb, None)
    r0 = sid * n_rows_tile

    def cp(i, _):
        pltpu.sync_copy(zbuf, acc_sh.at[pl.ds(r0 + i * zr, zr)])
        return _

    lax.fori_loop(0, n_rows_tile // zr, cp, None)


def _make_edge_kernel(E, N, gather, gidx_row=0):
    sidx_row = 1 - gidx_row if gather else 1
    ept = E // NS
    chunk = _chunk_size(ept)
    niter = ept // chunk
    half = _half_rows(N)
    n_rows_tile = half // NS
    zr = 400
    mesh = plsc.VectorSubcoreMesh(core_axis_name="c", subcore_axis_name="s")

    scratch = [
        pltpu.VMEM((NBUF, 2, chunk), jnp.int32),
        pltpu.VMEM((NBUF, chunk), jnp.int32),
        pltpu.VMEM((NBUF, chunk, LANES), F32) if gather
        else pltpu.VMEM((chunk, LANES), F32),
        pltpu.VMEM((zr, LANES), F32),
        pltpu.VMEM_SHARED((half + 8, LANES), F32),
        pltpu.SemaphoreType.DMA,
        pltpu.SemaphoreType.DMA,
    ]
    if gather:
        scratch.append(pltpu.SemaphoreType.DMA)

    @functools.partial(
        pl.kernel,
        out_type=jax.ShapeDtypeStruct((NC * half, LANES), F32),
        mesh=mesh,
        scratch_types=scratch,
        compiler_params=pltpu.CompilerParams(use_tc_tiling_on_sc=False),
    )
    def edge_kernel(*refs):
        if gather:
            (ei_hbm, table_hbm, out_hbm,
             idxb, redir, rows, zbuf, acc_sh, semI, semS, semG) = refs
        else:
            (ei_hbm, out_hbm,
             idxb, redir, rows, zbuf, acc_sh, semI, semS) = refs
        cid = lax.axis_index("c")
        sid = lax.axis_index("s")
        base_node = cid * half
        dummy = jnp.int32(half) + (lax.iota(jnp.int32, LANES) & 7)

        e0 = sid * ept

        def fire_idx(j, b):
            pltpu.async_copy(
                ei_hbm.at[:, pl.ds(e0 + j * chunk, chunk)], idxb.at[b], semI)

        def wait_idx(b):
            pltpu.make_async_copy(
                ei_hbm.at[:, pl.ds(0, chunk)], idxb.at[b], semI).wait()

        def redirect(b):
            for k in range(chunk // LANES):
                v = idxb[b, sidx_row, pl.ds(k * LANES, LANES)]
                local = v - base_node
                owned = (local >= 0) & (local < half)
                redir[b, pl.ds(k * LANES, LANES)] = jnp.where(
                    owned, local, dummy)

        def fire_gather(b):
            pltpu.async_copy(table_hbm.at[idxb.at[b, gidx_row]],
                             rows.at[b], semG)

        def wait_gather(b):
            pltpu.make_async_copy(table_hbm.at[pl.ds(0, chunk)],
                                  rows.at[b], semG).wait()

        def fire_scatter(b):
            src = rows.at[b] if gather else rows
            pltpu.async_copy(src, acc_sh.at[redir.at[b]], semS, add=True)

        def wait_scatter(b):
            src = rows.at[b] if gather else rows
            pltpu.make_async_copy(table_hbm.at[pl.ds(0, chunk)] if gather
                                  else out_hbm.at[pl.ds(0, chunk)],
                                  src, semS).wait()

        def step(j, b, first4, warm, alive, tail):
            if alive:
                wait_idx(b)
            if not first4:
                wait_scatter(b)
            if alive:
                redirect(b)
                if gather:
                    fire_gather(b)
            if warm:
                b2 = (b - 2) % NBUF
                if gather:
                    wait_gather(b2)
                fire_scatter(b2)
            if tail:
                fire_idx(j + 2, (b + 2) % NBUF)

        fire_idx(0, 0)
        fire_idx(1, 1)
        if not gather:
            ones16 = jnp.ones((LANES,), F32)

            def fill(i, _):
                rows[i, :] = ones16
                return _

            lax.fori_loop(0, chunk, fill, None)
        _zero_shared(acc_sh, zbuf, sid, n_rows_tile)
        plsc.subcore_barrier()

        for j in range(4):
            step(j, j, True, j >= 2, True, True)

        G = (niter - 2 - 4) // 4

        def body(g, _):
            j0 = 4 * (g + 1)
            for b in range(4):
                step(j0 + b, b, False, True, True, True)
            return _

        lax.fori_loop(0, G, body, None)

        for j in range(4 + 4 * G, niter + NBUF):
            step(j, j % NBUF, False, 2 <= j and j - 2 < niter,
                 j < niter, j + 2 < niter)

        plsc.subcore_barrier()
        r0 = sid * n_rows_tile
        pltpu.sync_copy(acc_sh.at[pl.ds(r0, n_rows_tile)],
                        out_hbm.at[pl.ds(base_node + r0, n_rows_tile)])

    return edge_kernel


def _make_edge_reduce_kernel(E, N):
    ept = E // (NC * NS)
    chunk = _chunk_size(ept)
    niter = ept // chunk
    mesh = plsc.VectorSubcoreMesh(core_axis_name="c", subcore_axis_name="s")

    @functools.partial(
        pl.kernel,
        out_type=jax.ShapeDtypeStruct((NC * NS, LANES), F32),
        mesh=mesh,
        scratch_types=[
            pltpu.VMEM((NBUF, 2, chunk), jnp.int32),
            pltpu.VMEM((NBUF, chunk, LANES), F32),
            pltpu.VMEM((NBUF, chunk, LANES), F32),
            pltpu.VMEM((1, LANES), F32),
            pltpu.SemaphoreType.DMA,
            pltpu.SemaphoreType.DMA,
        ],
        compiler_params=pltpu.CompilerParams(use_tc_tiling_on_sc=False),
    )
    def reduce_kernel(ei_hbm, q_hbm, dinv_hbm, out_hbm,
                      idxb, rows, rowsd, res_v, semI, semG):
        cid = lax.axis_index("c")
        sid = lax.axis_index("s")
        wid = cid * NS + sid
        e0 = wid * ept

        def fire_idx(j, b):
            pltpu.async_copy(
                ei_hbm.at[:, pl.ds(e0 + j * chunk, chunk)], idxb.at[b], semI)

        def wait_idx(b):
            pltpu.make_async_copy(
                ei_hbm.at[:, pl.ds(0, chunk)], idxb.at[b], semI).wait()

        def fire_gather(b):
            pltpu.async_copy(q_hbm.at[idxb.at[b, 0]], rows.at[b], semG)
            pltpu.async_copy(dinv_hbm.at[idxb.at[b, 1]], rowsd.at[b], semG)

        def wait_gather(b):
            pltpu.make_async_copy(q_hbm.at[pl.ds(0, chunk)],
                                  rows.at[b], semG).wait()
            pltpu.make_async_copy(q_hbm.at[pl.ds(0, chunk)],
                                  rowsd.at[b], semG).wait()

        def compute(b, acc):
            for i in range(chunk):
                acc = acc + rows[b, i, :] * rowsd[b, i, :]
            return acc

        def step(j, b, acc, warm, alive, tail):
            if alive:
                wait_idx(b)
                fire_gather(b)
            if warm:
                b2 = (b - 2) % NBUF
                wait_gather(b2)
                acc = compute(b2, acc)
            if tail:
                fire_idx(j + 2, (b + 2) % NBUF)
            return acc

        fire_idx(0, 0)
        fire_idx(1, 1)
        acc = jnp.zeros((LANES,), F32)
        for j in range(4):
            acc = step(j, j, acc, j >= 2, True, j + 2 < niter)

        G = (niter - 2 - 4) // 4

        def body(g, acc):
            j0 = 4 * (g + 1)
            for b in range(4):
                acc = step(j0 + b, b, acc, True, True, True)
            return acc

        acc = lax.fori_loop(0, G, body, acc)

        for j in range(4 + 4 * G, niter + 2):
            acc = step(j, j % NBUF, acc, 2 <= j and j - 2 < niter,
                       j < niter, j + 2 < niter)

        res_v[0, :] = acc
        pltpu.sync_copy(res_v, out_hbm.at[pl.ds(wid, 1)])

    return reduce_kernel


def _tc_matmul(x, W1, blk=1000):
    N, F = x.shape
    H = W1.shape[1]

    def body(x_ref, w_ref, h_ref):
        h_ref[...] = jnp.dot(x_ref[...], w_ref[...],
                             preferred_element_type=F32)

    return pl.pallas_call(
        body,
        grid=(N // blk,),
        in_specs=[
            pl.BlockSpec((blk, F), lambda i: (i, 0)),
            pl.BlockSpec((F, H), lambda i: (0, 0)),
        ],
        out_specs=pl.BlockSpec((blk, H), lambda i: (i, 0)),
        out_shape=jax.ShapeDtypeStruct((N, H), F32),
    )(x, W1)


def _tc_scale(h, deg16, blk=1000):
    N, H = h.shape

    def body(h_ref, deg_ref, g_ref, dinv_ref):
        dinv = 1.0 / jnp.sqrt(deg_ref[...] + 1.0)
        g_ref[...] = h_ref[...] * dinv
        dinv_ref[...] = dinv

    return pl.pallas_call(
        body,
        grid=(N // blk,),
        in_specs=[
            pl.BlockSpec((blk, H), lambda i: (i, 0)),
            pl.BlockSpec((blk, LANES), lambda i: (i, 0)),
        ],
        out_specs=[
            pl.BlockSpec((blk, H), lambda i: (i, 0)),
            pl.BlockSpec((blk, LANES), lambda i: (i, 0)),
        ],
        out_shape=[
            jax.ShapeDtypeStruct((N, H), F32),
            jax.ShapeDtypeStruct((N, LANES), F32),
        ],
    )(h, deg16)


def _tc_mid(agg, g, dinv16, b1, blk=1000):
    N, H = g.shape

    def body(agg_ref, g_ref, dinv_ref, b1_ref, q_ref, vself_ref, acc_ref):
        i = pl.program_id(0)

        @pl.when(i == 0)
        def _():
            acc_ref[...] = jnp.zeros_like(acc_ref)

        dinv = dinv_ref[...]
        r = jnp.maximum(
            dinv * (agg_ref[...] + g_ref[...]) + b1_ref[...], 0.0)
        q = dinv * r
        q_ref[...] = q
        acc_ref[...] += jnp.sum(dinv * q, axis=0, keepdims=True)

        @pl.when(i == pl.num_programs(0) - 1)
        def _():
            vself_ref[...] = acc_ref[...]

    return pl.pallas_call(
        body,
        grid=(N // blk,),
        in_specs=[
            pl.BlockSpec((blk, H), lambda i: (i, 0)),
            pl.BlockSpec((blk, H), lambda i: (i, 0)),
            pl.BlockSpec((blk, LANES), lambda i: (i, 0)),
            pl.BlockSpec((1, H), lambda i: (0, 0)),
        ],
        out_specs=[
            pl.BlockSpec((blk, H), lambda i: (i, 0)),
            pl.BlockSpec((1, H), lambda i: (0, 0)),
        ],
        out_shape=[
            jax.ShapeDtypeStruct((N, H), F32),
            jax.ShapeDtypeStruct((1, H), F32),
        ],
        scratch_shapes=[pltpu.VMEM((1, LANES), F32)],
    )(agg, g, dinv16, b1)


def _tc_fin2(vedge, vself, W2, b2, N):
    NW, H = vedge.shape
    EMB = W2.shape[1]

    def body(vedge_ref, vself_ref, w2_ref, b2_ref, out_ref):
        v = (jnp.sum(vedge_ref[...], axis=0, keepdims=True)
             + vself_ref[...])
        out_ref[...] = (jnp.dot(v, w2_ref[...],
                                preferred_element_type=F32) / N
                        + b2_ref[...])

    return pl.pallas_call(
        body,
        out_shape=jax.ShapeDtypeStruct((1, EMB), F32),
    )(vedge, vself, W2, b2)


def kernel(x, edge_index, W1, b1, W2, b2):
    N = x.shape[0]
    E = edge_index.shape[1]

    h = _tc_matmul(x, W1)
    deg16 = _make_edge_kernel(E, N, gather=False)(edge_index)
    g, dinv16 = _tc_scale(h, deg16)
    agg = _make_edge_kernel(E, N, gather=True, gidx_row=0)(edge_index, g)
    q, vself = _tc_mid(agg, g, dinv16, b1.reshape(1, -1))
    vedge = _make_edge_reduce_kernel(E, N)(edge_index, q, dinv16)
    return _tc_fin2(vedge, vself, W2, b2.reshape(1, -1), N)
